# baseline (speedup 1.0000x reference)
import functools

import jax
import jax.numpy as jnp
from jax import lax
from jax.experimental import pallas as pl
from jax.experimental.pallas import tpu as pltpu
from jax.experimental.pallas import tpu_sc as plsc

_NC = 2
_NS = 16
_NW = _NC * _NS

_CHUNK = 128
_K = 4
_BLK = 2048
_HALF = _BLK // 2


def _sc_gather_pack(table, patch_flat, off_rep, k, j_steps, n_nodes):
    chunk = _CHUNK
    feat = table.shape[1]
    per_w = j_steps * chunk
    pairs_out = _NW * per_w
    mesh = plsc.VectorSubcoreMesh(core_axis_name="c", subcore_axis_name="s")

    @functools.partial(
        pl.kernel,
        mesh=mesh,
        out_type=jax.ShapeDtypeStruct((pairs_out, feat), jnp.uint32),
        scratch_types=(
            [pltpu.VMEM((j_steps, 2, chunk), jnp.int32),
             pltpu.VMEM((16,), jnp.int32),
             pltpu.VMEM((2 * per_w, feat), jnp.float32),
             pltpu.VMEM((per_w, feat), jnp.uint32)]
            + [pltpu.SemaphoreType.DMA] * (2 * j_steps)
            + [pltpu.SemaphoreType.DMA]
        ),
    )
    def gather_k(table_hbm, patch_hbm, off_hbm, out_hbm,
                 idx_v, gv_v, rows_v, ebf_v, *sems):
        gsems, ssem = sems[:2 * j_steps], sems[2 * j_steps]
        wid = lax.axis_index("s") * _NC + lax.axis_index("c")
        pltpu.sync_copy(off_hbm.at[wid], gv_v)
        off_base = (16384 * k + 2048 * (wid // 4) + 256 * (wid % 4))
        for j in range(j_steps):
            pltpu.sync_copy(patch_hbm.at[pl.ds(off_base + 128 * j, chunk)],
                            idx_v.at[j, 0])
            pltpu.sync_copy(
                patch_hbm.at[pl.ds(off_base + 128 * j + _HALF, chunk)],
                idx_v.at[j, 1])
        goff = gv_v[...]
        for j in range(j_steps):
            for par in range(2):
                for s in range(chunk // 16):
                    sl = pl.ds(16 * s, 16)
                    idx_v[j, par, sl] = idx_v[j, par, sl] + goff
        base = wid * per_w
        gathers = [
            pltpu.async_copy(
                table_hbm.at[idx_v.at[j, par]],
                rows_v.at[pl.ds((2 * j + par) * chunk, chunk)],
                gsems[2 * j + par],
            )
            for j in range(j_steps)
            for par in range(2)
        ]
        half = jnp.uint32(0x8000)
        himask = jnp.uint32(0xFFFF0000)
        scatters = []
        for j in range(j_steps):
            gathers[2 * j].wait()
            gathers[2 * j + 1].wait()
            lo_slot = 2 * j * chunk
            hi_slot = lo_slot + chunk

            @plsc.parallel_loop(0, chunk, 1, unroll=4)
            def conv_pair(r, j=j, lo_slot=lo_slot, hi_slot=hi_slot):
                for s in range(feat // 16):
                    a = rows_v[lo_slot + r, pl.ds(16 * s, 16)]
                    b2 = rows_v[hi_slot + r, pl.ds(16 * s, 16)]
                    au = lax.bitcast_convert_type(a, jnp.uint32)
                    bu = lax.bitcast_convert_type(b2, jnp.uint32)
                    lo = lax.shift_right_logical(au + half, jnp.uint32(16))
                    hi = (bu + half) & himask
                    ebf_v[j * chunk + r, pl.ds(16 * s, 16)] = lo | hi

            scatters.append(
                pltpu.async_copy(
                    ebf_v.at[pl.ds(j * chunk, chunk)],
                    out_hbm.at[pl.ds(base + j * chunk, chunk)],
                    ssem,
                )
            )
        for s in scatters:
            s.wait()

    return gather_k(table, patch_flat, off_rep)


def _film_body(e_ref, x_ref, w_ref, b_ref, out_ref):
    feat = e_ref.shape[-1]
    eu = e_ref[...]
    ea = lax.bitcast_convert_type(eu << jnp.uint32(16), jnp.float32)
    eb = lax.bitcast_convert_type(eu & jnp.uint32(0xFFFF0000), jnp.float32)
    w = w_ref[...]
    bb = b_ref[...]
    ha = jnp.dot(ea.astype(jnp.bfloat16), w,
                 preferred_element_type=jnp.float32) + bb
    hb = jnp.dot(eb.astype(jnp.bfloat16), w,
                 preferred_element_type=jnp.float32) + bb
    xx = x_ref[...]
    out_ref[:_HALF, :] = xx[:_HALF] * ha[:, :feat] + ha[:, feat:]
    out_ref[_HALF:, :] = xx[_HALF:] * hb[:, :feat] + hb[:, feat:]


def _film_body_chained(e_ref, x_ref, w_ref, b_ref, buf_ref, out_ref):
    del buf_ref
    _film_body(e_ref, x_ref, w_ref, b_ref, out_ref)


def _film_chunk(e_k, x2, W16, b2, buf, k, rows, feat):
    pairs = e_k.shape[0]
    nb = pairs // _HALF
    e_spec = pl.BlockSpec((_HALF, feat), lambda i: (i, 0))
    x_spec = pl.BlockSpec((_BLK, feat), lambda i: (k * nb + i, 0))
    w_spec = pl.BlockSpec((feat, 2 * feat), lambda i: (0, 0))
    b_spec = pl.BlockSpec((1, 2 * feat), lambda i: (0, 0))
    out_spec = pl.BlockSpec((_BLK, feat), lambda i: (k * nb + i, 0))
    out_shape = jax.ShapeDtypeStruct((rows, feat), jnp.float32)
    if buf is None:
        return pl.pallas_call(
            _film_body,
            grid=(nb,),
            in_specs=[e_spec, x_spec, w_spec, b_spec],
            out_specs=out_spec,
            out_shape=out_shape,
        )(e_k, x2, W16, b2)
    buf_spec = pl.BlockSpec((8, feat), lambda i: (0, 0))
    return pl.pallas_call(
        _film_body_chained,
        grid=(nb,),
        in_specs=[e_spec, x_spec, w_spec, b_spec, buf_spec],
        out_specs=out_spec,
        out_shape=out_shape,
        input_output_aliases={4: 0},
    )(e_k, x2, W16, b2, buf)


def kernel(x, patch_idx, group_idx, embeddings, W, b):
    batch, patch, feat = x.shape
    n_groups, n_nodes, _ = embeddings.shape
    rows = batch * patch

    table = embeddings.reshape(n_groups * n_nodes, feat)
    patch_flat = patch_idx.astype(jnp.int32).reshape(rows)
    j_steps = rows // (2 * _K * _NW * _CHUNK)
    wids = jnp.arange(_NW, dtype=jnp.int32)
    samp = 4 * jnp.arange(_K, dtype=jnp.int32)[:, None] + wids[None, :] // 8
    off_rep = jnp.broadcast_to(
        (group_idx.astype(jnp.int32)[samp] * jnp.int32(n_nodes))[:, :, None],
        (_K, _NW, 16)).astype(jnp.int32)

    e_chunks = [
        _sc_gather_pack(table, patch_flat, off_rep[k], k, j_steps, n_nodes)
        for k in range(_K)
    ]

    W16 = W.astype(jnp.bfloat16)
    x2 = x.reshape(rows, feat)
    b2 = b.reshape(1, 2 * feat)
    buf = None
    for k in range(_K):
        buf = _film_chunk(e_chunks[k], x2, W16, b2, buf, k, rows, feat)
    return buf.reshape(batch, patch, feat)


# speedup vs baseline: 1.0878x; 1.0878x over previous
import functools

import jax
import jax.numpy as jnp
from jax import lax
from jax.experimental import pallas as pl
from jax.experimental.pallas import tpu as pltpu
from jax.experimental.pallas import tpu_sc as plsc

_NC = 2
_NS = 16
_NW = _NC * _NS

_CHUNK = 128
_K = 4
_BLK = 2048
_HALF = _BLK // 2


def _sc_gather_pack(table, idx4):
    nw, j_steps, _, chunk = idx4.shape
    pairs_out = nw * j_steps * chunk
    feat = table.shape[1]
    per_w = j_steps * chunk
    mesh = plsc.VectorSubcoreMesh(core_axis_name="c", subcore_axis_name="s")

    @functools.partial(
        pl.kernel,
        mesh=mesh,
        out_type=jax.ShapeDtypeStruct((pairs_out, feat), jnp.uint32),
        scratch_types=(
            [pltpu.VMEM((j_steps, 2, chunk), jnp.int32),
             pltpu.VMEM((2 * per_w, feat), jnp.float32),
             pltpu.VMEM((per_w, feat), jnp.uint32)]
            + [pltpu.SemaphoreType.DMA] * (2 * j_steps)
            + [pltpu.SemaphoreType.DMA]
        ),
    )
    def gather_k(table_hbm, idx_hbm, out_hbm, idx_v, rows_v, ebf_v, *sems):
        gsems, ssem = sems[:2 * j_steps], sems[2 * j_steps]
        wid = lax.axis_index("s") * _NC + lax.axis_index("c")
        pltpu.sync_copy(idx_hbm.at[wid], idx_v)
        base = wid * per_w
        gathers = [
            pltpu.async_copy(
                table_hbm.at[idx_v.at[j, par]],
                rows_v.at[pl.ds((2 * j + par) * chunk, chunk)],
                gsems[2 * j + par],
            )
            for j in range(j_steps)
            for par in range(2)
        ]
        half = jnp.uint32(0x8000)
        himask = jnp.uint32(0xFFFF0000)
        scatters = []
        for j in range(j_steps):
            gathers[2 * j].wait()
            gathers[2 * j + 1].wait()
            lo_slot = 2 * j * chunk
            hi_slot = lo_slot + chunk

            @plsc.parallel_loop(0, chunk, 1, unroll=4)
            def conv_pair(r, j=j, lo_slot=lo_slot, hi_slot=hi_slot):
                for s in range(feat // 16):
                    a = rows_v[lo_slot + r, pl.ds(16 * s, 16)]
                    b2 = rows_v[hi_slot + r, pl.ds(16 * s, 16)]
                    au = lax.bitcast_convert_type(a, jnp.uint32)
                    bu = lax.bitcast_convert_type(b2, jnp.uint32)
                    lo = lax.shift_right_logical(au + half, jnp.uint32(16))
                    hi = (bu + half) & himask
                    ebf_v[j * chunk + r, pl.ds(16 * s, 16)] = lo | hi

            scatters.append(
                pltpu.async_copy(
                    ebf_v.at[pl.ds(j * chunk, chunk)],
                    out_hbm.at[pl.ds(base + j * chunk, chunk)],
                    ssem,
                )
            )
        for s in scatters:
            s.wait()

    return gather_k(table, idx4)


def _film_body(e_ref, x_ref, w_ref, b_ref, out_ref):
    feat = e_ref.shape[-1]
    eu = e_ref[...]
    ea = lax.bitcast_convert_type(eu << jnp.uint32(16), jnp.float32)
    eb = lax.bitcast_convert_type(eu & jnp.uint32(0xFFFF0000), jnp.float32)
    w = w_ref[...]
    bb = b_ref[...]
    ha = jnp.dot(ea.astype(jnp.bfloat16), w,
                 preferred_element_type=jnp.float32) + bb
    hb = jnp.dot(eb.astype(jnp.bfloat16), w,
                 preferred_element_type=jnp.float32) + bb
    xx = x_ref[...]
    out_ref[:_HALF, :] = xx[:_HALF] * ha[:, :feat] + ha[:, feat:]
    out_ref[_HALF:, :] = xx[_HALF:] * hb[:, :feat] + hb[:, feat:]


def _film_body_chained(e_ref, x_ref, w_ref, b_ref, buf_ref, out_ref):
    del buf_ref
    _film_body(e_ref, x_ref, w_ref, b_ref, out_ref)


def _film_chunk(e_k, x2, W16, b2, buf, k, rows, feat):
    pairs = e_k.shape[0]
    nb = pairs // _HALF
    e_spec = pl.BlockSpec((_HALF, feat), lambda i: (i, 0))
    x_spec = pl.BlockSpec((_BLK, feat), lambda i: (k * nb + i, 0))
    w_spec = pl.BlockSpec((feat, 2 * feat), lambda i: (0, 0))
    b_spec = pl.BlockSpec((1, 2 * feat), lambda i: (0, 0))
    out_spec = pl.BlockSpec((_BLK, feat), lambda i: (k * nb + i, 0))
    out_shape = jax.ShapeDtypeStruct((rows, feat), jnp.float32)
    if buf is None:
        return pl.pallas_call(
            _film_body,
            grid=(nb,),
            in_specs=[e_spec, x_spec, w_spec, b_spec],
            out_specs=out_spec,
            out_shape=out_shape,
        )(e_k, x2, W16, b2)
    buf_spec = pl.BlockSpec((8, feat), lambda i: (0, 0))
    return pl.pallas_call(
        _film_body_chained,
        grid=(nb,),
        in_specs=[e_spec, x_spec, w_spec, b_spec, buf_spec],
        out_specs=out_spec,
        out_shape=out_shape,
        input_output_aliases={4: 0},
    )(e_k, x2, W16, b2, buf)


def kernel(x, patch_idx, group_idx, embeddings, W, b):
    batch, patch, feat = x.shape
    n_groups, n_nodes, _ = embeddings.shape
    rows = batch * patch

    table = embeddings.reshape(n_groups * n_nodes, feat)
    flat_idx = (group_idx.astype(jnp.int32)[:, None] * n_nodes
                + patch_idx.astype(jnp.int32))
    flat2 = flat_idx.reshape(rows // _BLK, 2, _HALF)
    j_steps = rows // (2 * _K * _NW * _CHUNK)
    ev = flat2[:, 0, :].reshape(_K, _NW, j_steps, 1, _CHUNK)
    od = flat2[:, 1, :].reshape(_K, _NW, j_steps, 1, _CHUNK)
    idx5 = jnp.concatenate([ev, od], axis=3)

    e_chunks = [_sc_gather_pack(table, idx5[k]) for k in range(_K)]

    W16 = W.astype(jnp.bfloat16)
    x2 = x.reshape(rows, feat)
    b2 = b.reshape(1, 2 * feat)
    buf = None
    for k in range(_K):
        buf = _film_chunk(e_chunks[k], x2, W16, b2, buf, k, rows, feat)
    return buf.reshape(batch, patch, feat)
